# Initial kernel scaffold; baseline (speedup 1.0000x reference)
import jax
import jax.numpy as jnp
from jax.experimental import pallas as pl


def kernel(x, edge_index, W1l, b1, W1r, W2l, b2, W2r):
    raise NotImplementedError("write your pallas kernel here")


# speedup vs baseline: 4.6118x; 4.6118x over previous
import functools

import jax
import jax.numpy as jnp
from jax import lax
from jax.experimental import pallas as pl
from jax.experimental.pallas import tpu as pltpu
from jax.experimental.pallas import tpu_sc as plsc

NC = 2
NS = 16
NW = NC * NS
CHUNK = 80
CNT_W = 128
ROW_BLK = 1000


def _dot_t(a, w):
    return lax.dot_general(a, w, (((1,), (1,)), ((), ())),
                           preferred_element_type=jnp.float32)


def _mm_pre_body(x_ref, wl_ref, wr_ref, b_ref, xl_ref, xrb_ref):
    x = x_ref[...]
    xl_ref[...] = _dot_t(x, wl_ref[...])
    xrb_ref[...] = _dot_t(x, wr_ref[...]) + b_ref[...]


def _mm_pre(x, wl, wr, b):
    n, d = x.shape
    grid = (n // ROW_BLK,)
    return pl.pallas_call(
        _mm_pre_body,
        grid=grid,
        in_specs=[
            pl.BlockSpec((ROW_BLK, d), lambda i: (i, 0)),
            pl.BlockSpec((d, d), lambda i: (0, 0)),
            pl.BlockSpec((d, d), lambda i: (0, 0)),
            pl.BlockSpec((1, d), lambda i: (0, 0)),
        ],
        out_specs=[
            pl.BlockSpec((ROW_BLK, d), lambda i: (i, 0)),
            pl.BlockSpec((ROW_BLK, d), lambda i: (i, 0)),
        ],
        out_shape=[
            jax.ShapeDtypeStruct((n, d), jnp.float32),
            jax.ShapeDtypeStruct((n, d), jnp.float32),
        ],
    )(x, wl, wr, b.reshape(1, d))


def _mid_body(p_ref, cnt_ref, xrb_ref, wl_ref, wr_ref, b_ref,
              hl_ref, hrb_ref):
    acc = p_ref[0] + p_ref[1]
    cnt = cnt_ref[0, :, 0:1] + cnt_ref[1, :, 0:1]
    h = acc / jnp.maximum(cnt, 1.0) + xrb_ref[...]
    h = jnp.maximum(h, 0.0)
    hl_ref[...] = _dot_t(h, wl_ref[...])
    hrb_ref[...] = _dot_t(h, wr_ref[...]) + b_ref[...]


def _mid(p, cnt, xrb, wl, wr, b):
    n, d = xrb.shape
    grid = (n // ROW_BLK,)
    return pl.pallas_call(
        _mid_body,
        grid=grid,
        in_specs=[
            pl.BlockSpec((NC, ROW_BLK, d), lambda i: (0, i, 0)),
            pl.BlockSpec((NC, ROW_BLK, CNT_W), lambda i: (0, i, 0)),
            pl.BlockSpec((ROW_BLK, d), lambda i: (i, 0)),
            pl.BlockSpec((d, d), lambda i: (0, 0)),
            pl.BlockSpec((d, d), lambda i: (0, 0)),
            pl.BlockSpec((1, d), lambda i: (0, 0)),
        ],
        out_specs=[
            pl.BlockSpec((ROW_BLK, d), lambda i: (i, 0)),
            pl.BlockSpec((ROW_BLK, d), lambda i: (i, 0)),
        ],
        out_shape=[
            jax.ShapeDtypeStruct((n, d), jnp.float32),
            jax.ShapeDtypeStruct((n, d), jnp.float32),
        ],
    )(p, cnt, xrb, wl, wr, b.reshape(1, d))


def _final_body(p_ref, cnt_ref, hrb_ref, out_ref):
    acc = p_ref[0] + p_ref[1]
    cnt = cnt_ref[0, :, 0:1] + cnt_ref[1, :, 0:1]
    out_ref[...] = acc / jnp.maximum(cnt, 1.0) + hrb_ref[...]


def _final(p, cnt, hrb):
    n, d = hrb.shape
    grid = (n // ROW_BLK,)
    return pl.pallas_call(
        _final_body,
        grid=grid,
        in_specs=[
            pl.BlockSpec((NC, ROW_BLK, d), lambda i: (0, i, 0)),
            pl.BlockSpec((NC, ROW_BLK, CNT_W), lambda i: (0, i, 0)),
            pl.BlockSpec((ROW_BLK, d), lambda i: (i, 0)),
        ],
        out_specs=pl.BlockSpec((ROW_BLK, d), lambda i: (i, 0)),
        out_shape=jax.ShapeDtypeStruct((n, d), jnp.float32),
    )(p, cnt, hrb)


def _make_segsum(n, d, nchunk):
    mesh = plsc.VectorSubcoreMesh(core_axis_name="c", subcore_axis_name="s")
    rows_per = n // NS

    def body(xl_hbm, src_hbm, dst_hbm, zrow_hbm,
             out_hbm, idx_s, idx_d, rows, acc_sh, sem):
        c = lax.axis_index("c")
        s = lax.axis_index("s")
        wid = c * NS + s
        base = wid * nchunk * CHUNK

        sl = pl.ds(s * rows_per, rows_per)
        pltpu.sync_copy(zrow_hbm.at[sl], acc_sh.at[sl])
        plsc.subcore_barrier()

        def chunk(j, carry):
            off = base + j * CHUNK
            pltpu.sync_copy(src_hbm.at[pl.ds(off, CHUNK)], idx_s)
            pltpu.sync_copy(dst_hbm.at[pl.ds(off, CHUNK)], idx_d)
            pltpu.async_copy(xl_hbm.at[idx_s], rows, sem).wait()
            pltpu.sync_copy(rows, acc_sh.at[idx_d], add=True)
            return carry

        lax.fori_loop(0, nchunk, chunk, 0)
        plsc.subcore_barrier()
        pltpu.sync_copy(acc_sh.at[sl], out_hbm.at[c, sl])

    return functools.partial(
        pl.kernel, mesh=mesh,
        out_type=[jax.ShapeDtypeStruct((NC, n, d), jnp.float32)],
        scratch_types=[
            pltpu.VMEM((CHUNK,), jnp.int32),
            pltpu.VMEM((CHUNK,), jnp.int32),
            pltpu.VMEM((CHUNK, d), jnp.float32),
            pltpu.VMEM_SHARED((n, d), jnp.float32),
            pltpu.SemaphoreType.DMA,
        ],
    )(body)


def _make_cnt(n, nchunk):
    mesh = plsc.VectorSubcoreMesh(core_axis_name="c", subcore_axis_name="s")
    rows_per = n // NS

    def body(dst_hbm, zcnt_hbm, ones_hbm,
             cnt_hbm, idx_d, ones_v, cnt_sh):
        c = lax.axis_index("c")
        s = lax.axis_index("s")
        wid = c * NS + s
        base = wid * nchunk * CHUNK

        pltpu.sync_copy(ones_hbm, ones_v)
        sl = pl.ds(s * rows_per, rows_per)
        pltpu.sync_copy(zcnt_hbm.at[sl], cnt_sh.at[sl])
        plsc.subcore_barrier()

        def chunk(j, carry):
            pltpu.sync_copy(dst_hbm.at[pl.ds(base + j * CHUNK, CHUNK)], idx_d)
            pltpu.sync_copy(ones_v, cnt_sh.at[idx_d], add=True)
            return carry

        lax.fori_loop(0, nchunk, chunk, 0)
        plsc.subcore_barrier()
        pltpu.sync_copy(cnt_sh.at[sl], cnt_hbm.at[c, sl])

    return functools.partial(
        pl.kernel, mesh=mesh,
        out_type=[jax.ShapeDtypeStruct((NC, n, CNT_W), jnp.float32)],
        scratch_types=[
            pltpu.VMEM((CHUNK,), jnp.int32),
            pltpu.VMEM((CHUNK, CNT_W), jnp.float32),
            pltpu.VMEM_SHARED((n, CNT_W), jnp.float32),
        ],
    )(body)


def kernel(x, edge_index, W1l, b1, W1r, W2l, b2, W2r):
    n, d = x.shape
    e = edge_index.shape[1]
    assert e % NW == 0 and n % NS == 0
    per_tile = e // NW
    assert per_tile % CHUNK == 0
    nchunk = per_tile // CHUNK

    blk = NS * 8
    n_pad = (n + blk - 1) // blk * blk

    ei = edge_index.astype(jnp.int32)
    src_f = ei[0]
    dst_f = ei[1]
    zrow = jnp.zeros((n_pad, d), jnp.float32)
    zcnt = jnp.zeros((n_pad, CNT_W), jnp.float32)
    ones = jnp.ones((CHUNK, CNT_W), jnp.float32)

    seg = _make_segsum(n_pad, d, nchunk)
    cntk = _make_cnt(n_pad, nchunk)

    (cnt,) = cntk(dst_f, zcnt, ones)
    xl, xrb = _mm_pre(x, W1l, W1r, b1)
    (p1,) = seg(xl, src_f, dst_f, zrow)
    hl, hrb = _mid(p1[:, :n], cnt[:, :n], xrb, W2l, W2r, b2)
    (p2,) = seg(hl, src_f, dst_f, zrow)
    return _final(p2[:, :n], cnt[:, :n], hrb)


# speedup vs baseline: 8.0186x; 1.7387x over previous
import functools

import jax
import jax.numpy as jnp
from jax import lax
from jax.experimental import pallas as pl
from jax.experimental.pallas import tpu as pltpu
from jax.experimental.pallas import tpu_sc as plsc

NC = 2
NS = 16
NW = NC * NS
CHUNK = 80
CNT_W = 16
ROW_BLK = 1000


def _dot_t(a, w):
    return lax.dot_general(a, w, (((1,), (1,)), ((), ())),
                           preferred_element_type=jnp.float32)


def _mm_pre_body(x_ref, wl_ref, wr_ref, b_ref, xl_ref, xrb_ref):
    x = x_ref[...]
    xl_ref[...] = _dot_t(x, wl_ref[...])
    xrb_ref[...] = _dot_t(x, wr_ref[...]) + b_ref[...]


def _mm_pre(x, wl, wr, b):
    n, d = x.shape
    grid = (n // ROW_BLK,)
    return pl.pallas_call(
        _mm_pre_body,
        grid=grid,
        in_specs=[
            pl.BlockSpec((ROW_BLK, d), lambda i: (i, 0)),
            pl.BlockSpec((d, d), lambda i: (0, 0)),
            pl.BlockSpec((d, d), lambda i: (0, 0)),
            pl.BlockSpec((1, d), lambda i: (0, 0)),
        ],
        out_specs=[
            pl.BlockSpec((ROW_BLK, d), lambda i: (i, 0)),
            pl.BlockSpec((ROW_BLK, d), lambda i: (i, 0)),
        ],
        out_shape=[
            jax.ShapeDtypeStruct((n, d), jnp.float32),
            jax.ShapeDtypeStruct((n, d), jnp.float32),
        ],
    )(x, wl, wr, b.reshape(1, d))


def _mid_body(p_ref, cnt_ref, xrb_ref, wl_ref, wr_ref, b_ref,
              hl_ref, hrb_ref):
    acc = p_ref[0] + p_ref[1]
    cnt = cnt_ref[0, :, 0:1] + cnt_ref[1, :, 0:1]
    h = acc / jnp.maximum(cnt, 1.0) + xrb_ref[...]
    h = jnp.maximum(h, 0.0)
    hl_ref[...] = _dot_t(h, wl_ref[...])
    hrb_ref[...] = _dot_t(h, wr_ref[...]) + b_ref[...]


def _mid(p, cnt, xrb, wl, wr, b):
    n, d = xrb.shape
    grid = (n // ROW_BLK,)
    return pl.pallas_call(
        _mid_body,
        grid=grid,
        in_specs=[
            pl.BlockSpec((NC, ROW_BLK, d), lambda i: (0, i, 0)),
            pl.BlockSpec((NC, ROW_BLK, CNT_W), lambda i: (0, i, 0)),
            pl.BlockSpec((ROW_BLK, d), lambda i: (i, 0)),
            pl.BlockSpec((d, d), lambda i: (0, 0)),
            pl.BlockSpec((d, d), lambda i: (0, 0)),
            pl.BlockSpec((1, d), lambda i: (0, 0)),
        ],
        out_specs=[
            pl.BlockSpec((ROW_BLK, d), lambda i: (i, 0)),
            pl.BlockSpec((ROW_BLK, d), lambda i: (i, 0)),
        ],
        out_shape=[
            jax.ShapeDtypeStruct((n, d), jnp.float32),
            jax.ShapeDtypeStruct((n, d), jnp.float32),
        ],
    )(p, cnt, xrb, wl, wr, b.reshape(1, d))


def _final_body(p_ref, cnt_ref, hrb_ref, out_ref):
    acc = p_ref[0] + p_ref[1]
    cnt = cnt_ref[0, :, 0:1] + cnt_ref[1, :, 0:1]
    out_ref[...] = acc / jnp.maximum(cnt, 1.0) + hrb_ref[...]


def _final(p, cnt, hrb):
    n, d = hrb.shape
    grid = (n // ROW_BLK,)
    return pl.pallas_call(
        _final_body,
        grid=grid,
        in_specs=[
            pl.BlockSpec((NC, ROW_BLK, d), lambda i: (0, i, 0)),
            pl.BlockSpec((NC, ROW_BLK, CNT_W), lambda i: (0, i, 0)),
            pl.BlockSpec((ROW_BLK, d), lambda i: (i, 0)),
        ],
        out_specs=pl.BlockSpec((ROW_BLK, d), lambda i: (i, 0)),
        out_shape=jax.ShapeDtypeStruct((n, d), jnp.float32),
    )(p, cnt, hrb)


def _make_segsum(n, d, nchunk):
    mesh = plsc.VectorSubcoreMesh(core_axis_name="c", subcore_axis_name="s")
    rows_per = n // NS

    assert nchunk % 2 == 1
    per_tile = nchunk * CHUNK

    def body(xl_hbm, src_hbm, dst_hbm, zrow_hbm,
             out_hbm, idx_s, idx_d, rows0, rows1, acc_sh, sem0, sem1):
        c = lax.axis_index("c")
        s = lax.axis_index("s")
        wid = c * NS + s

        pltpu.sync_copy(src_hbm.at[pl.ds(wid * per_tile, per_tile)], idx_s)
        pltpu.sync_copy(dst_hbm.at[wid], idx_d)
        sl = pl.ds(s * rows_per, rows_per)
        pltpu.sync_copy(zrow_hbm.at[sl], acc_sh.at[sl])
        plsc.subcore_barrier()

        def gather(j, rows, sem):
            return pltpu.async_copy(
                xl_hbm.at[idx_s.at[pl.ds(j * CHUNK, CHUNK)]], rows, sem)

        def gwait(rows, sem):
            pltpu.make_async_copy(
                xl_hbm.at[idx_s.at[pl.ds(0, CHUNK)]], rows, sem).wait()

        def scatter(j, rows):
            pltpu.sync_copy(rows, acc_sh.at[idx_d.at[j]], add=True)

        gather(0, rows0, sem0)

        def pair(k, carry):
            j = 2 * k
            gwait(rows0, sem0)
            gather(j + 1, rows1, sem1)
            scatter(j, rows0)
            gwait(rows1, sem1)
            gather(j + 2, rows0, sem0)
            scatter(j + 1, rows1)
            return carry

        lax.fori_loop(0, (nchunk - 1) // 2, pair, 0)
        gwait(rows0, sem0)
        scatter(nchunk - 1, rows0)

        plsc.subcore_barrier()
        pltpu.sync_copy(acc_sh.at[sl], out_hbm.at[c, sl])

    return functools.partial(
        pl.kernel, mesh=mesh,
        out_type=[jax.ShapeDtypeStruct((NC, n, d), jnp.float32)],
        scratch_types=[
            pltpu.VMEM((per_tile,), jnp.int32),
            pltpu.VMEM((nchunk, CHUNK), jnp.int32),
            pltpu.VMEM((CHUNK, d), jnp.float32),
            pltpu.VMEM((CHUNK, d), jnp.float32),
            pltpu.VMEM_SHARED((n, d), jnp.float32),
            pltpu.SemaphoreType.DMA,
            pltpu.SemaphoreType.DMA,
        ],
    )(body)


def _make_cnt(n, nchunk):
    mesh = plsc.VectorSubcoreMesh(core_axis_name="c", subcore_axis_name="s")
    rows_per = n // NS
    assert nchunk % 2 == 1

    def body(dst_hbm, zcnt_hbm, ones_hbm,
             cnt_hbm, idx_d, ones_v, cnt_sh, sem0, sem1):
        c = lax.axis_index("c")
        s = lax.axis_index("s")
        wid = c * NS + s

        pltpu.sync_copy(dst_hbm.at[wid], idx_d)
        pltpu.sync_copy(ones_hbm, ones_v)
        sl = pl.ds(s * rows_per, rows_per)
        pltpu.sync_copy(zcnt_hbm.at[sl], cnt_sh.at[sl])
        plsc.subcore_barrier()

        def start(j, sem):
            return pltpu.async_copy(ones_v, cnt_sh.at[idx_d.at[j]], sem,
                                    add=True)

        def wait(j, sem):
            pltpu.make_async_copy(ones_v, cnt_sh.at[idx_d.at[j]], sem).wait()

        start(0, sem0)
        start(1, sem1)

        def pair(k, carry):
            j = 2 * k
            wait(j, sem0)
            start(j + 2, sem0)
            wait(j + 1, sem1)
            start(j + 3, sem1)
            return carry

        lax.fori_loop(0, (nchunk - 3) // 2, pair, 0)
        wait(nchunk - 3, sem0)
        start(nchunk - 1, sem0)
        wait(nchunk - 2, sem1)
        wait(nchunk - 1, sem0)

        plsc.subcore_barrier()
        pltpu.sync_copy(cnt_sh.at[sl], cnt_hbm.at[c, sl])

    return functools.partial(
        pl.kernel, mesh=mesh,
        out_type=[jax.ShapeDtypeStruct((NC, n, 128), jnp.float32)],
        scratch_types=[
            pltpu.VMEM((nchunk, CHUNK), jnp.int32),
            pltpu.VMEM((CHUNK, 128), jnp.float32),
            pltpu.VMEM_SHARED((n, 128), jnp.float32),
            pltpu.SemaphoreType.DMA,
            pltpu.SemaphoreType.DMA,
        ],
    )(body)


def kernel(x, edge_index, W1l, b1, W1r, W2l, b2, W2r):
    n, d = x.shape
    e = edge_index.shape[1]
    assert e % NW == 0 and n % NS == 0
    per_tile = e // NW
    assert per_tile % CHUNK == 0
    nchunk = per_tile // CHUNK

    blk = NS * 8
    n_pad = (n + blk - 1) // blk * blk

    ei = edge_index.astype(jnp.int32)
    src_f = ei[0]
    dst3 = ei[1].reshape(NW, nchunk, CHUNK)
    zrow = jnp.zeros((n_pad, d), jnp.float32)
    zcnt = jnp.zeros((n_pad, 128), jnp.float32)
    ones = jnp.ones((CHUNK, 128), jnp.float32)

    seg = _make_segsum(n_pad, d, nchunk)
    cntk = _make_cnt(n_pad, nchunk)

    (cnt_full,) = cntk(dst3, zcnt, ones)
    cnt = cnt_full[:, :, :CNT_W]
    xl, xrb = _mm_pre(x, W1l, W1r, b1)
    (p1,) = seg(xl, src_f, dst3, zrow)
    hl, hrb = _mid(p1[:, :n], cnt[:, :n], xrb, W2l, W2r, b2)
    (p2,) = seg(hl, src_f, dst3, zrow)
    return _final(p2[:, :n], cnt[:, :n], hrb)


# speedup vs baseline: 8.1613x; 1.0178x over previous
import functools

import jax
import jax.numpy as jnp
from jax import lax
from jax.experimental import pallas as pl
from jax.experimental.pallas import tpu as pltpu
from jax.experimental.pallas import tpu_sc as plsc

NC = 2
NS = 16
NW = NC * NS
CHUNK = 80
CNT_W = 16
ROW_BLK = 1000


def _dot_t(a, w):
    return lax.dot_general(a, w, (((1,), (1,)), ((), ())),
                           preferred_element_type=jnp.float32)


def _mm_pre_body(x_ref, wl_ref, wr_ref, b_ref, xl_ref, xrb_ref):
    x = x_ref[...]
    xl_ref[...] = _dot_t(x, wl_ref[...])
    xrb_ref[...] = _dot_t(x, wr_ref[...]) + b_ref[...]


def _mm_pre(x, wl, wr, b, blk):
    n, d = x.shape
    grid = (n // blk,)
    return pl.pallas_call(
        _mm_pre_body,
        grid=grid,
        in_specs=[
            pl.BlockSpec((blk, d), lambda i: (i, 0)),
            pl.BlockSpec((d, d), lambda i: (0, 0)),
            pl.BlockSpec((d, d), lambda i: (0, 0)),
            pl.BlockSpec((1, d), lambda i: (0, 0)),
        ],
        out_specs=[
            pl.BlockSpec((blk, d), lambda i: (i, 0)),
            pl.BlockSpec((blk, d), lambda i: (i, 0)),
        ],
        out_shape=[
            jax.ShapeDtypeStruct((n, d), jnp.float32),
            jax.ShapeDtypeStruct((n, d), jnp.float32),
        ],
    )(x, wl, wr, b.reshape(1, d))


def _mid_body(p_ref, cnt_ref, xrb_ref, wl_ref, wr_ref, b_ref,
              hl_ref, hrb_ref):
    acc = p_ref[0] + p_ref[1]
    cnt = cnt_ref[0, :, 0:1] + cnt_ref[1, :, 0:1]
    h = acc / jnp.maximum(cnt, 1.0) + xrb_ref[...]
    h = jnp.maximum(h, 0.0)
    hl_ref[...] = _dot_t(h, wl_ref[...])
    hrb_ref[...] = _dot_t(h, wr_ref[...]) + b_ref[...]


def _mid(p, cnt, xrb, wl, wr, b, blk):
    n, d = xrb.shape
    grid = (n // blk,)
    return pl.pallas_call(
        _mid_body,
        grid=grid,
        in_specs=[
            pl.BlockSpec((NC, blk, d), lambda i: (0, i, 0)),
            pl.BlockSpec((NC, blk, d), lambda i: (0, i, 0)),
            pl.BlockSpec((blk, d), lambda i: (i, 0)),
            pl.BlockSpec((d, d), lambda i: (0, 0)),
            pl.BlockSpec((d, d), lambda i: (0, 0)),
            pl.BlockSpec((1, d), lambda i: (0, 0)),
        ],
        out_specs=[
            pl.BlockSpec((blk, d), lambda i: (i, 0)),
            pl.BlockSpec((blk, d), lambda i: (i, 0)),
        ],
        out_shape=[
            jax.ShapeDtypeStruct((n, d), jnp.float32),
            jax.ShapeDtypeStruct((n, d), jnp.float32),
        ],
    )(p, cnt, xrb, wl, wr, b.reshape(1, d))


def _final_body(p_ref, cnt_ref, hrb_ref, out_ref):
    acc = p_ref[0] + p_ref[1]
    cnt = cnt_ref[0, :, 0:1] + cnt_ref[1, :, 0:1]
    out_ref[...] = acc / jnp.maximum(cnt, 1.0) + hrb_ref[...]


def _final(p, cnt, hrb, n_out, blk):
    d = hrb.shape[1]
    grid = (n_out // blk,)
    return pl.pallas_call(
        _final_body,
        grid=grid,
        in_specs=[
            pl.BlockSpec((NC, blk, d), lambda i: (0, i, 0)),
            pl.BlockSpec((NC, blk, d), lambda i: (0, i, 0)),
            pl.BlockSpec((blk, d), lambda i: (i, 0)),
        ],
        out_specs=pl.BlockSpec((blk, d), lambda i: (i, 0)),
        out_shape=jax.ShapeDtypeStruct((n_out, d), jnp.float32),
    )(p, cnt, hrb)


def _make_segsum(n, d, nchunk):
    mesh = plsc.VectorSubcoreMesh(core_axis_name="c", subcore_axis_name="s")
    rows_per = n // NS

    assert nchunk % 2 == 1
    per_tile = nchunk * CHUNK

    def body(xl_hbm, src_hbm, dst_hbm, zrow_hbm,
             out_hbm, idx_s, idx_d, rows0, rows1, acc_sh, sem0, sem1):
        c = lax.axis_index("c")
        s = lax.axis_index("s")
        wid = c * NS + s

        pltpu.sync_copy(src_hbm.at[pl.ds(wid * per_tile, per_tile)], idx_s)
        pltpu.sync_copy(dst_hbm.at[wid], idx_d)
        sl = pl.ds(s * rows_per, rows_per)
        pltpu.sync_copy(zrow_hbm.at[sl], acc_sh.at[sl])
        plsc.subcore_barrier()

        def gather(j, rows, sem):
            return pltpu.async_copy(
                xl_hbm.at[idx_s.at[pl.ds(j * CHUNK, CHUNK)]], rows, sem)

        def gwait(rows, sem):
            pltpu.make_async_copy(
                xl_hbm.at[idx_s.at[pl.ds(0, CHUNK)]], rows, sem).wait()

        def scatter(j, rows):
            pltpu.sync_copy(rows, acc_sh.at[idx_d.at[j]], add=True)

        gather(0, rows0, sem0)

        def pair(k, carry):
            j = 2 * k
            gwait(rows0, sem0)
            gather(j + 1, rows1, sem1)
            scatter(j, rows0)
            gwait(rows1, sem1)
            gather(j + 2, rows0, sem0)
            scatter(j + 1, rows1)
            return carry

        lax.fori_loop(0, (nchunk - 1) // 2, pair, 0)
        gwait(rows0, sem0)
        scatter(nchunk - 1, rows0)

        plsc.subcore_barrier()
        pltpu.sync_copy(acc_sh.at[sl], out_hbm.at[c, sl])

    return functools.partial(
        pl.kernel, mesh=mesh,
        out_type=[jax.ShapeDtypeStruct((NC, n, d), jnp.float32)],
        scratch_types=[
            pltpu.VMEM((per_tile,), jnp.int32),
            pltpu.VMEM((nchunk, CHUNK), jnp.int32),
            pltpu.VMEM((CHUNK, d), jnp.float32),
            pltpu.VMEM((CHUNK, d), jnp.float32),
            pltpu.VMEM_SHARED((n, d), jnp.float32),
            pltpu.SemaphoreType.DMA,
            pltpu.SemaphoreType.DMA,
        ],
    )(body)


def _make_cnt(n, nchunk):
    mesh = plsc.VectorSubcoreMesh(core_axis_name="c", subcore_axis_name="s")
    rows_per = n // NS
    assert nchunk % 2 == 1

    def body(dst_hbm, zcnt_hbm, ones_hbm,
             cnt_hbm, idx_d, ones_v, cnt_sh, sem0, sem1):
        c = lax.axis_index("c")
        s = lax.axis_index("s")
        wid = c * NS + s

        pltpu.sync_copy(dst_hbm.at[wid], idx_d)
        pltpu.sync_copy(ones_hbm, ones_v)
        sl = pl.ds(s * rows_per, rows_per)
        pltpu.sync_copy(zcnt_hbm.at[sl], cnt_sh.at[sl])
        plsc.subcore_barrier()

        def start(j, sem):
            return pltpu.async_copy(ones_v, cnt_sh.at[idx_d.at[j]], sem,
                                    add=True)

        def wait(j, sem):
            pltpu.make_async_copy(ones_v, cnt_sh.at[idx_d.at[j]], sem).wait()

        start(0, sem0)
        start(1, sem1)

        def pair(k, carry):
            j = 2 * k
            wait(j, sem0)
            start(j + 2, sem0)
            wait(j + 1, sem1)
            start(j + 3, sem1)
            return carry

        lax.fori_loop(0, (nchunk - 3) // 2, pair, 0)
        wait(nchunk - 3, sem0)
        start(nchunk - 1, sem0)
        wait(nchunk - 2, sem1)
        wait(nchunk - 1, sem0)

        plsc.subcore_barrier()
        pltpu.sync_copy(cnt_sh.at[sl], cnt_hbm.at[c, sl])

    return functools.partial(
        pl.kernel, mesh=mesh,
        out_type=[jax.ShapeDtypeStruct((NC, n, 128), jnp.float32)],
        scratch_types=[
            pltpu.VMEM((nchunk, CHUNK), jnp.int32),
            pltpu.VMEM((CHUNK, 128), jnp.float32),
            pltpu.VMEM_SHARED((n, 128), jnp.float32),
            pltpu.SemaphoreType.DMA,
            pltpu.SemaphoreType.DMA,
        ],
    )(body)


def kernel(x, edge_index, W1l, b1, W1r, W2l, b2, W2r):
    n, d = x.shape
    e = edge_index.shape[1]
    assert e % NW == 0 and n % NS == 0
    per_tile = e // NW
    assert per_tile % CHUNK == 0
    nchunk = per_tile // CHUNK

    blk = NS * 8
    n_pad = (n + blk - 1) // blk * blk

    ei = edge_index.astype(jnp.int32)
    src_f = ei[0]
    dst3 = ei[1].reshape(NW, nchunk, CHUNK)
    zrow = jnp.zeros((n_pad, d), jnp.float32)
    ones = jnp.ones((CHUNK, 128), jnp.float32)
    x_pad = jnp.pad(x, ((0, n_pad - n), (0, 0)))

    seg = _make_segsum(n_pad, d, nchunk)
    cntk = _make_cnt(n_pad, nchunk)

    blk_tc = n_pad // 16
    (cnt_full,) = cntk(dst3, zrow, ones)
    xl, xrb = _mm_pre(x_pad, W1l, W1r, b1, blk_tc)
    (p1,) = seg(xl, src_f, dst3, zrow)
    hl, hrb = _mid(p1, cnt_full, xrb, W2l, W2r, b2, blk_tc)
    (p2,) = seg(hl, src_f, dst3, zrow)
    return _final(p2, cnt_full, hrb, n, ROW_BLK)
